# Initial kernel scaffold; baseline (speedup 1.0000x reference)
import jax
import jax.numpy as jnp
from jax.experimental import pallas as pl


def kernel(x, edge_idx, fc0_w, fc0_b, convs_w, fc1_w, fc1_b):
    raise NotImplementedError("write your pallas kernel here")


# speedup vs baseline: 6.3490x; 6.3490x over previous
import functools
import math

import jax
import jax.numpy as jnp
from jax import lax
from jax.experimental import pallas as pl
from jax.experimental.pallas import tpu as pltpu
from jax.experimental.pallas import tpu_sc as plsc

N = 10000
E = 320000
NFEAT = 128
NHID = 128
NCLASS = 64
NLAYERS = 8
LAMDA = 0.5
ALPHA = 0.1

NC = 2
NS = 16
NW = NC * NS
CHUNK = 128
CPT = -(-E // (NW * CHUNK))
EPT = CPT * CHUNK
E_PAD = EPT * NW
NACC = N + 8
RPT = 632
RPT_LAST_OUT = N - (NS - 1) * RPT
RPT_LAST_ZERO = NACC - (NS - 1) * RPT
LANES = 16


def _spmm_sc_body(g_hbm, col_hbm, row_hbm, out_hbm, cidx_v, ridx_v, rows_v, zbuf_v, acc_sh, sem):
    c = lax.axis_index("c")
    s = lax.axis_index("s")
    wid = c * NS + s

    zeros = jnp.zeros((LANES,), jnp.float32)

    def zero_row(i, _):
        for j in range(NHID // LANES):
            zbuf_v[i, pl.ds(j * LANES, LANES)] = zeros
        return 0

    lax.fori_loop(0, CHUNK, zero_row, 0)

    def zero_slab(base, nrows):
        nfull = nrows // CHUNK
        for k in range(nfull):
            pltpu.sync_copy(zbuf_v, acc_sh.at[pl.ds(base + k * CHUNK, CHUNK)])
        rem = nrows - nfull * CHUNK
        if rem:
            pltpu.sync_copy(
                zbuf_v.at[pl.ds(0, rem)], acc_sh.at[pl.ds(base + nfull * CHUNK, rem)]
            )

    @pl.when(s < NS - 1)
    def _():
        zero_slab(s * RPT, RPT)

    @pl.when(s == NS - 1)
    def _():
        zero_slab((NS - 1) * RPT, RPT_LAST_ZERO)

    plsc.subcore_barrier()

    ebase = wid * EPT

    def body(i, _):
        base = ebase + i * CHUNK
        pltpu.sync_copy(col_hbm.at[pl.ds(base, CHUNK)], cidx_v)
        pltpu.sync_copy(row_hbm.at[pl.ds(base, CHUNK)], ridx_v)
        pltpu.async_copy(g_hbm.at[cidx_v], rows_v, sem).wait()
        pltpu.sync_copy(rows_v, acc_sh.at[ridx_v], add=True)
        return 0

    lax.fori_loop(0, CPT, body, 0)

    plsc.subcore_barrier()

    @pl.when(s < NS - 1)
    def _():
        ob = s * RPT
        pltpu.sync_copy(acc_sh.at[pl.ds(ob, RPT)], out_hbm.at[c, pl.ds(ob, RPT)])

    @pl.when(s == NS - 1)
    def _():
        ob = (NS - 1) * RPT
        pltpu.sync_copy(
            acc_sh.at[pl.ds(ob, RPT_LAST_OUT)], out_hbm.at[c, pl.ds(ob, RPT_LAST_OUT)]
        )


@functools.cache
def _get_spmm_sc():
    return pl.kernel(
        _spmm_sc_body,
        out_type=jax.ShapeDtypeStruct((NC, N, NHID), jnp.float32),
        mesh=plsc.VectorSubcoreMesh(
            core_axis_name="c", subcore_axis_name="s", num_cores=NC, num_subcores=NS
        ),
        scratch_types=[
            pltpu.VMEM((CHUNK,), jnp.int32),
            pltpu.VMEM((CHUNK,), jnp.int32),
            pltpu.VMEM((CHUNK, NHID), jnp.float32),
            pltpu.VMEM((CHUNK, NHID), jnp.float32),
            pltpu.VMEM_SHARED((NACC, NHID), jnp.float32),
            pltpu.SemaphoreType.DMA,
        ],
    )


def _spmm_sc(g, col_p, row_p):
    return _get_spmm_sc()(g, col_p, row_p)


_BT = 1000


def _pre_body(x_ref, w_ref, b_ref, dis_ref, h0_ref, g_ref):
    h = jnp.dot(x_ref[...], w_ref[...], preferred_element_type=jnp.float32)
    h = jnp.maximum(h + b_ref[...], 0.0)
    h0_ref[...] = h
    g_ref[...] = h * dis_ref[...]


def _dense_pre(x, w_t, b, dis):
    return pl.pallas_call(
        _pre_body,
        grid=(N // _BT,),
        in_specs=[
            pl.BlockSpec((_BT, NFEAT), lambda i: (i, 0)),
            pl.BlockSpec((NFEAT, NHID), lambda i: (0, 0)),
            pl.BlockSpec((1, NHID), lambda i: (0, 0)),
            pl.BlockSpec((_BT, 1), lambda i: (i, 0)),
        ],
        out_specs=[
            pl.BlockSpec((_BT, NHID), lambda i: (i, 0)),
            pl.BlockSpec((_BT, NHID), lambda i: (i, 0)),
        ],
        out_shape=[
            jax.ShapeDtypeStruct((N, NHID), jnp.float32),
            jax.ShapeDtypeStruct((N, NHID), jnp.float32),
        ],
    )(x, w_t, b, dis)


def _layer_body(theta, p0_ref, p1_ref, g_ref, h0_ref, dis_ref, w_ref, h_ref, gn_ref):
    dis = dis_ref[...]
    hi = dis * (p0_ref[...] + p1_ref[...] + g_ref[...])
    support = (1.0 - ALPHA) * hi + ALPHA * h0_ref[...]
    out = theta * jnp.dot(support, w_ref[...], preferred_element_type=jnp.float32)
    out = out + (1.0 - theta) * support
    h = jnp.maximum(out, 0.0)
    h_ref[...] = h
    gn_ref[...] = h * dis


def _dense_layer(theta, p0, p1, g, h0, dis, w):
    return pl.pallas_call(
        functools.partial(_layer_body, theta),
        grid=(N // _BT,),
        in_specs=[
            pl.BlockSpec((_BT, NHID), lambda i: (i, 0)),
            pl.BlockSpec((_BT, NHID), lambda i: (i, 0)),
            pl.BlockSpec((_BT, NHID), lambda i: (i, 0)),
            pl.BlockSpec((_BT, NHID), lambda i: (i, 0)),
            pl.BlockSpec((_BT, 1), lambda i: (i, 0)),
            pl.BlockSpec((NHID, NHID), lambda i: (0, 0)),
        ],
        out_specs=[
            pl.BlockSpec((_BT, NHID), lambda i: (i, 0)),
            pl.BlockSpec((_BT, NHID), lambda i: (i, 0)),
        ],
        out_shape=[
            jax.ShapeDtypeStruct((N, NHID), jnp.float32),
            jax.ShapeDtypeStruct((N, NHID), jnp.float32),
        ],
    )(p0, p1, g, h0, dis, w)


def _final_body(h_ref, w_ref, b_ref, o_ref):
    o = jnp.dot(h_ref[...], w_ref[...], preferred_element_type=jnp.float32)
    o_ref[...] = o + b_ref[...]


def _dense_final(h, w_t, b):
    return pl.pallas_call(
        _final_body,
        grid=(N // _BT,),
        in_specs=[
            pl.BlockSpec((_BT, NHID), lambda i: (i, 0)),
            pl.BlockSpec((NHID, NCLASS), lambda i: (0, 0)),
            pl.BlockSpec((1, NCLASS), lambda i: (0, 0)),
        ],
        out_specs=pl.BlockSpec((_BT, NCLASS), lambda i: (i, 0)),
        out_shape=jax.ShapeDtypeStruct((N, NCLASS), jnp.float32),
    )(h, w_t, b)


def kernel(x, edge_idx, fc0_w, fc0_b, convs_w, fc1_w, fc1_b):
    row = edge_idx[0].astype(jnp.int32)
    col = edge_idx[1].astype(jnp.int32)
    self_m = row == col
    w_e = jnp.where(self_m, 0.0, 1.0).astype(jnp.float32)
    deg = jnp.zeros((N,), jnp.float32).at[row].add(w_e) + 1.0
    dis = lax.rsqrt(deg).reshape(N, 1)

    row_r = jnp.where(self_m, N, row)
    row_p = jnp.concatenate([row_r, jnp.full((E_PAD - E,), N, jnp.int32)])
    col_p = jnp.concatenate([col, jnp.zeros((E_PAD - E,), jnp.int32)])

    h0, g = _dense_pre(x, fc0_w.T, fc0_b.reshape(1, NHID), dis)
    h = h0
    for i in range(NLAYERS):
        theta = math.log(LAMDA / (i + 1) + 1.0)
        p = _spmm_sc(g, col_p, row_p)
        h, g = _dense_layer(theta, p[0], p[1], g, h0, dis, convs_w[i])
    return _dense_final(h, fc1_w.T, fc1_b.reshape(1, NCLASS))
